# baseline (speedup 1.0000x reference)
import functools

import jax
import jax.numpy as jnp
from jax import lax
from jax.experimental import pallas as pl
from jax.experimental.pallas import tpu as pltpu
from jax.experimental.pallas import tpu_sc as plsc

NUM_CORES = 2
NUM_SUBCORES = 16
NUM_WORKERS = NUM_CORES * NUM_SUBCORES
BATCH = 16384
DIM = 128
BW = BATCH // NUM_WORKERS
CHUNK = 128
NCHUNK = BW // CHUNK


def _body(h_hbm, t_hbm, r_hbm, e_hbm, rel_hbm, w_hbm, out_hbm,
          hidx, tidx, ridx, hbuf, tbuf, rbuf, wbuf, dtr, wtr, outb, sem):
    wid = lax.axis_index("s") * NUM_CORES + lax.axis_index("c")

    pltpu.sync_copy(h_hbm.at[wid], hidx)
    pltpu.sync_copy(t_hbm.at[wid], tidx)
    pltpu.sync_copy(r_hbm.at[wid], ridx)

    lanes = lax.iota(jnp.int32, 16)

    def chunk_body(k, carry):
        c1 = pltpu.async_copy(e_hbm.at[hidx.at[k]], hbuf, sem)
        c2 = pltpu.async_copy(e_hbm.at[tidx.at[k]], tbuf, sem)
        c3 = pltpu.async_copy(rel_hbm.at[ridx.at[k]], rbuf, sem)
        c4 = pltpu.async_copy(w_hbm.at[ridx.at[k]], wbuf, sem)
        c1.wait()
        c2.wait()
        c3.wait()
        c4.wait()

        def group_body(g, carry2):
            rowv = g * 16 + lanes
            s1 = jnp.zeros((16,), jnp.float32)
            s2 = jnp.zeros((16,), jnp.float32)
            for jd in range(DIM):
                colv = jnp.full((16,), jd, jnp.int32)
                hv = plsc.load_gather(hbuf, [rowv, colv])
                tv = plsc.load_gather(tbuf, [rowv, colv])
                wv = plsc.load_gather(wbuf, [rowv, colv])
                dv = hv - tv
                dtr[jd] = dv
                wtr[jd] = wv
                s1 = s1 + dv * wv
                s2 = s2 + wv * wv
            coeff = s1 / jnp.maximum(s2, 1e-24)
            acc = jnp.zeros((16,), jnp.float32)
            for jd in range(DIM):
                colv = jnp.full((16,), jd, jnp.int32)
                rv = plsc.load_gather(rbuf, [rowv, colv])
                acc = acc + jnp.abs(dtr[jd] + rv - coeff * wtr[jd])
            base = pl.multiple_of(k * CHUNK + g * 16, 16)
            outb[pl.ds(base, 16)] = acc
            return carry2

        lax.fori_loop(0, CHUNK // 16, group_body, 0)
        return carry

    lax.fori_loop(0, NCHUNK, chunk_body, 0)
    pltpu.sync_copy(outb, out_hbm.at[pl.ds(pl.multiple_of(wid * BW, 8), BW)])


@jax.jit
def kernel(h, r, t, E, R, W):
    mesh = plsc.VectorSubcoreMesh(core_axis_name="c", subcore_axis_name="s")
    kfn = pl.kernel(
        _body,
        out_type=jax.ShapeDtypeStruct((BATCH,), jnp.float32),
        mesh=mesh,
        compiler_params=pltpu.CompilerParams(needs_layout_passes=False),
        scratch_types=[
            pltpu.VMEM((NCHUNK, CHUNK), jnp.int32),
            pltpu.VMEM((NCHUNK, CHUNK), jnp.int32),
            pltpu.VMEM((NCHUNK, CHUNK), jnp.int32),
            pltpu.VMEM((CHUNK, DIM), jnp.float32),
            pltpu.VMEM((CHUNK, DIM), jnp.float32),
            pltpu.VMEM((CHUNK, DIM), jnp.float32),
            pltpu.VMEM((CHUNK, DIM), jnp.float32),
            pltpu.VMEM((DIM, 16), jnp.float32),
            pltpu.VMEM((DIM, 16), jnp.float32),
            pltpu.VMEM((BW,), jnp.float32),
            pltpu.SemaphoreType.DMA,
        ],
    )
    h2 = h.reshape(NUM_WORKERS, NCHUNK, CHUNK)
    t2 = t.reshape(NUM_WORKERS, NCHUNK, CHUNK)
    r2 = r.reshape(NUM_WORKERS, NCHUNK, CHUNK)
    return kfn(h2, t2, r2, E, R, W)


# speedup vs baseline: 4.1616x; 4.1616x over previous
import functools

import jax
import jax.numpy as jnp
from jax import lax
from jax.experimental import pallas as pl
from jax.experimental.pallas import tpu as pltpu
from jax.experimental.pallas import tpu_sc as plsc

NUM_CORES = 2
NUM_SUBCORES = 16
NUM_WORKERS = NUM_CORES * NUM_SUBCORES
BATCH = 16384
DIM = 128
NJ = DIM // 16
BW = BATCH // NUM_WORKERS
CHUNK = 128
NCHUNK = BW // CHUNK


def _body(h_hbm, t_hbm, r_hbm, e_hbm, rel_hbm, w_hbm, out_hbm,
          hidx, tidx, ridx, hbuf, tbuf, rbuf, wbuf, outb, sem):
    wid = lax.axis_index("s") * NUM_CORES + lax.axis_index("c")

    pltpu.sync_copy(h_hbm.at[wid], hidx)
    pltpu.sync_copy(t_hbm.at[wid], tidx)
    pltpu.sync_copy(r_hbm.at[wid], ridx)

    lanes = lax.iota(jnp.int32, 16)

    def chunk_body(k, carry):
        c1 = pltpu.async_copy(e_hbm.at[hidx.at[k]], hbuf, sem)
        c2 = pltpu.async_copy(e_hbm.at[tidx.at[k]], tbuf, sem)
        c3 = pltpu.async_copy(rel_hbm.at[ridx.at[k]], rbuf, sem)
        c4 = pltpu.async_copy(w_hbm.at[ridx.at[k]], wbuf, sem)
        c1.wait()
        c2.wait()
        c3.wait()
        c4.wait()

        def group_body(g, carry2):
            outv = jnp.zeros((16,), jnp.float32)
            for rr in range(16):
                i = g * 16 + rr
                d = []
                w = []
                s1v = jnp.zeros((16,), jnp.float32)
                s2v = jnp.zeros((16,), jnp.float32)
                for j in range(NJ):
                    sl = pl.ds(j * 16, 16)
                    dv = hbuf[i, sl] - tbuf[i, sl]
                    wv = wbuf[i, sl]
                    d.append(dv)
                    w.append(wv)
                    s1v = s1v + dv * wv
                    s2v = s2v + wv * wv
                s1 = jnp.broadcast_to(jnp.sum(s1v), (16,))
                s2 = jnp.broadcast_to(jnp.sum(s2v), (16,))
                coeff = s1 / jnp.maximum(s2, 1e-24)
                accv = jnp.zeros((16,), jnp.float32)
                for j in range(NJ):
                    rv = rbuf[i, pl.ds(j * 16, 16)]
                    accv = accv + jnp.abs(d[j] + rv - coeff * w[j])
                acc = jnp.broadcast_to(jnp.sum(accv), (16,))
                outv = jnp.where(lanes == rr, acc, outv)
            base = pl.multiple_of(k * CHUNK + g * 16, 16)
            outb[pl.ds(base, 16)] = outv
            return carry2

        lax.fori_loop(0, CHUNK // 16, group_body, 0)
        return carry

    lax.fori_loop(0, NCHUNK, chunk_body, 0)
    pltpu.sync_copy(outb, out_hbm.at[pl.ds(pl.multiple_of(wid * BW, 8), BW)])


@jax.jit
def kernel(h, r, t, E, R, W):
    mesh = plsc.VectorSubcoreMesh(core_axis_name="c", subcore_axis_name="s")
    kfn = pl.kernel(
        _body,
        out_type=jax.ShapeDtypeStruct((BATCH,), jnp.float32),
        mesh=mesh,
        compiler_params=pltpu.CompilerParams(needs_layout_passes=False),
        scratch_types=[
            pltpu.VMEM((NCHUNK, CHUNK), jnp.int32),
            pltpu.VMEM((NCHUNK, CHUNK), jnp.int32),
            pltpu.VMEM((NCHUNK, CHUNK), jnp.int32),
            pltpu.VMEM((CHUNK, DIM), jnp.float32),
            pltpu.VMEM((CHUNK, DIM), jnp.float32),
            pltpu.VMEM((CHUNK, DIM), jnp.float32),
            pltpu.VMEM((CHUNK, DIM), jnp.float32),
            pltpu.VMEM((BW,), jnp.float32),
            pltpu.SemaphoreType.DMA,
        ],
    )
    h2 = h.reshape(NUM_WORKERS, NCHUNK, CHUNK)
    t2 = t.reshape(NUM_WORKERS, NCHUNK, CHUNK)
    r2 = r.reshape(NUM_WORKERS, NCHUNK, CHUNK)
    return kfn(h2, t2, r2, E, R, W)


# speedup vs baseline: 4.7335x; 1.1374x over previous
import functools

import jax
import jax.numpy as jnp
from jax import lax
from jax.experimental import pallas as pl
from jax.experimental.pallas import tpu as pltpu
from jax.experimental.pallas import tpu_sc as plsc

NUM_CORES = 2
NUM_SUBCORES = 16
NUM_WORKERS = NUM_CORES * NUM_SUBCORES
BATCH = 16384
DIM = 128
NJ = DIM // 16
BW = BATCH // NUM_WORKERS
CHUNK = 64
NCHUNK = BW // CHUNK


def _body(h_hbm, t_hbm, r_hbm, e_hbm, rel_hbm, w_hbm, out_hbm,
          hidx, tidx, ridx, bufs0, bufs1, outb, sem0, sem1):
    wid = lax.axis_index("s") * NUM_CORES + lax.axis_index("c")

    pltpu.sync_copy(h_hbm.at[wid], hidx)
    pltpu.sync_copy(t_hbm.at[wid], tidx)
    pltpu.sync_copy(r_hbm.at[wid], ridx)

    lanes = lax.iota(jnp.int32, 16)

    def issue(k, bufs, sem):
        pltpu.async_copy(e_hbm.at[hidx.at[k]], bufs[0], sem)
        pltpu.async_copy(e_hbm.at[tidx.at[k]], bufs[1], sem)
        pltpu.async_copy(rel_hbm.at[ridx.at[k]], bufs[2], sem)
        pltpu.async_copy(w_hbm.at[ridx.at[k]], bufs[3], sem)

    def drain(bufs, sem):
        for b in bufs:
            pltpu.make_async_copy(e_hbm.at[pl.ds(0, CHUNK)], b, sem).wait()

    def compute(k, bufs):
        hbuf, tbuf, rbuf, wbuf = bufs

        def group_body(g, carry2):
            outv = jnp.zeros((16,), jnp.float32)
            for rr in range(16):
                i = g * 16 + rr
                d = []
                w = []
                s1v = jnp.zeros((16,), jnp.float32)
                s2v = jnp.zeros((16,), jnp.float32)
                for j in range(NJ):
                    sl = pl.ds(j * 16, 16)
                    dv = hbuf[i, sl] - tbuf[i, sl]
                    wv = wbuf[i, sl]
                    d.append(dv)
                    w.append(wv)
                    s1v = s1v + dv * wv
                    s2v = s2v + wv * wv
                s1 = jnp.broadcast_to(jnp.sum(s1v), (16,))
                s2 = jnp.broadcast_to(jnp.sum(s2v), (16,))
                coeff = s1 / jnp.maximum(s2, 1e-24)
                accv = jnp.zeros((16,), jnp.float32)
                for j in range(NJ):
                    rv = rbuf[i, pl.ds(j * 16, 16)]
                    accv = accv + jnp.abs(d[j] + rv - coeff * w[j])
                acc = jnp.broadcast_to(jnp.sum(accv), (16,))
                outv = jnp.where(lanes == rr, acc, outv)
            base = pl.multiple_of(k * CHUNK + g * 16, 16)
            outb[pl.ds(base, 16)] = outv
            return carry2

        lax.fori_loop(0, CHUNK // 16, group_body, 0)

    issue(0, bufs0, sem0)

    def pair_body(p, carry):
        k0 = 2 * p
        issue(k0 + 1, bufs1, sem1)
        drain(bufs0, sem0)
        compute(k0, bufs0)

        @pl.when(k0 + 2 < NCHUNK)
        def _():
            issue(k0 + 2, bufs0, sem0)

        drain(bufs1, sem1)
        compute(k0 + 1, bufs1)
        return carry

    lax.fori_loop(0, NCHUNK // 2, pair_body, 0)
    pltpu.sync_copy(outb, out_hbm.at[pl.ds(pl.multiple_of(wid * BW, 8), BW)])


@jax.jit
def kernel(h, r, t, E, R, W):
    mesh = plsc.VectorSubcoreMesh(core_axis_name="c", subcore_axis_name="s")
    buf = pltpu.VMEM((CHUNK, DIM), jnp.float32)
    kfn = pl.kernel(
        _body,
        out_type=jax.ShapeDtypeStruct((BATCH,), jnp.float32),
        mesh=mesh,
        compiler_params=pltpu.CompilerParams(needs_layout_passes=False),
        scratch_types=[
            pltpu.VMEM((NCHUNK, CHUNK), jnp.int32),
            pltpu.VMEM((NCHUNK, CHUNK), jnp.int32),
            pltpu.VMEM((NCHUNK, CHUNK), jnp.int32),
            [buf, buf, buf, buf],
            [buf, buf, buf, buf],
            pltpu.VMEM((BW,), jnp.float32),
            pltpu.SemaphoreType.DMA,
            pltpu.SemaphoreType.DMA,
        ],
    )
    h2 = h.reshape(NUM_WORKERS, NCHUNK, CHUNK)
    t2 = t.reshape(NUM_WORKERS, NCHUNK, CHUNK)
    r2 = r.reshape(NUM_WORKERS, NCHUNK, CHUNK)
    return kfn(h2, t2, r2, E, R, W)
